# Initial kernel scaffold; baseline (speedup 1.0000x reference)
import jax
import jax.numpy as jnp
from jax.experimental import pallas as pl


def kernel(x, edge_index, W1l, W1r, b1, W2l, W2r, b2, Wfc, bfc):
    raise NotImplementedError("write your pallas kernel here")


# speedup vs baseline: 11.9414x; 11.9414x over previous
import functools

import jax
import jax.numpy as jnp
from jax import lax
from jax.experimental import pallas as pl
from jax.experimental.pallas import tpu as pltpu
from jax.experimental.pallas import tpu_sc as plsc

N = 10000
E = 320000
F = 128
NCLS = 2
NPAD = 10240

NC = 2
NS = 16
NW = NC * NS
C = 125
EPT = E // NW
CPT = EPT // C
SL = 4
NSLAB = CPT // SL
ONES = 128
RPT = NPAD // NS
DPT = NPAD // NS

BM = 1000
GRID = N // BM


def _make_segsum(with_deg: bool):
  out_type = [jax.ShapeDtypeStruct((NC, NPAD, F), jnp.float32)]
  if with_deg:
    out_type.append(jax.ShapeDtypeStruct((NC, 1, NPAD), jnp.float32))
  mesh = plsc.VectorSubcoreMesh(core_axis_name="c", subcore_axis_name="s")
  scratch_types = [
      pltpu.VMEM((2, SL, 2, C), jnp.int32),
      pltpu.VMEM((2, C, F), jnp.float32),
      pltpu.VMEM_SHARED((NPAD, F), jnp.float32),
      pltpu.SemaphoreType.DMA,
      pltpu.SemaphoreType.DMA,
      pltpu.SemaphoreType.DMA,
      pltpu.VMEM((400,), jnp.float32),
  ]
  if with_deg:
    scratch_types += [
        pltpu.VMEM((ONES,), jnp.float32),
        pltpu.VMEM_SHARED((NPAD,), jnp.float32),
        pltpu.SemaphoreType.DMA,
    ]

  def body(x_hbm, idx_hbm, zx_hbm, zd_hbm, *rest):
    if with_deg:
      agg_out, deg_out = rest[0], rest[1]
      (idxs_v, rows_v, acc_sh, sem, semi, sems, dsc_v, ones_v, deg_sh,
       semd) = rest[2:]
    else:
      agg_out = rest[0]
      idxs_v, rows_v, acc_sh, sem, semi, sems, dsc_v = rest[1:]

    cid = lax.axis_index("c")
    sid = lax.axis_index("s")
    wid = sid * NC + cid

    rbase = sid * RPT
    pltpu.sync_copy(zx_hbm.at[pl.ds(rbase, RPT)], acc_sh.at[pl.ds(rbase, RPT)])
    if with_deg:
      dbase = sid * DPT
      pltpu.sync_copy(zd_hbm.at[pl.ds(dbase, DPT)],
                      deg_sh.at[pl.ds(dbase, DPT)])
      for j in range(ONES // 16):
        ones_v[pl.ds(j * 16, 16)] = jnp.ones((16,), jnp.float32)

    slab0 = wid * NSLAB
    pltpu.sync_copy(idx_hbm.at[slab0], idxs_v.at[0])
    plsc.subcore_barrier()

    def slab(s, carry):
      p = lax.rem(s, 2)

      @pl.when(s < NSLAB - 1)
      def _prefetch():
        pltpu.async_copy(idx_hbm.at[slab0 + s + 1], idxs_v.at[1 - p], semi)

      pltpu.async_copy(x_hbm.at[idxs_v.at[p, 0, 0]], rows_v.at[0], sem)

      def chunk(q, c2):
        qm = lax.rem(q, 2)

        @pl.when(q < SL - 1)
        def _fire():
          @pl.when(q >= 1)
          def _free():
            pltpu.make_async_copy(rows_v.at[1 - qm],
                                  acc_sh.at[idxs_v.at[p, q - 1, 1]],
                                  sems).wait()
          pltpu.async_copy(x_hbm.at[idxs_v.at[p, q + 1, 0]], rows_v.at[1 - qm],
                           sem)

        pltpu.make_async_copy(x_hbm.at[idxs_v.at[p, q, 0]], rows_v.at[qm],
                              sem).wait()
        pltpu.async_copy(rows_v.at[qm], acc_sh.at[idxs_v.at[p, q, 1]], sems,
                         add=True)
        if with_deg:
          pltpu.async_copy(ones_v.at[pl.ds(0, C)],
                           deg_sh.at[idxs_v.at[p, q, 1]], semd, add=True)
        return c2

      lax.fori_loop(0, SL, chunk, 0)
      pltpu.make_async_copy(rows_v.at[0], acc_sh.at[idxs_v.at[p, 0, 1]],
                            sems).wait()
      pltpu.make_async_copy(rows_v.at[0], acc_sh.at[idxs_v.at[p, 0, 1]],
                            sems).wait()

      @pl.when(s < NSLAB - 1)
      def _drain():
        pltpu.make_async_copy(idx_hbm.at[slab0 + s + 1], idxs_v.at[1 - p],
                              semi).wait()
      return carry

    lax.fori_loop(0, NSLAB, slab, 0)
    if with_deg:
      def draind(q, c2):
        pltpu.make_async_copy(ones_v.at[pl.ds(0, C)],
                              deg_sh.at[idxs_v.at[0, 0, 1]], semd).wait()
        return c2
      lax.fori_loop(0, CPT, draind, 0)
    plsc.subcore_barrier()

    pltpu.sync_copy(acc_sh.at[pl.ds(rbase, RPT)],
                    agg_out.at[cid, pl.ds(rbase, RPT)])
    if with_deg:
      pltpu.sync_copy(deg_sh.at[pl.ds(dbase, DPT)],
                      deg_out.at[cid, 0, pl.ds(dbase, DPT)])

  return pl.kernel(body, mesh=mesh, out_type=out_type,
                   scratch_types=scratch_types)


_segsum_deg = _make_segsum(True)
_segsum = _make_segsum(False)


def _tcr_body(x_ref, w_ref, b_ref, out_ref):
  out_ref[...] = (jnp.dot(x_ref[...], w_ref[...],
                          preferred_element_type=jnp.float32) + b_ref[...])


def _tc1_body(aggp_ref, degt_ref, xr_ref, wl_ref, out_ref):
  d = degt_ref[...]
  deg = d[:, 0:1] + d[:, 1:2]
  inv = 1.0 / jnp.maximum(deg, 1.0)
  mean = (aggp_ref[0] + aggp_ref[1]) * inv
  h = (jnp.dot(mean, wl_ref[...], preferred_element_type=jnp.float32)
       + xr_ref[...])
  out_ref[...] = jnp.maximum(h, 0.0)


def _tc2_body(aggp_ref, degt_ref, hr_ref, wl_ref, wfc_ref, bfc_ref, out_ref):
  d = degt_ref[...]
  deg = d[:, 0:1] + d[:, 1:2]
  inv = 1.0 / jnp.maximum(deg, 1.0)
  mean = (aggp_ref[0] + aggp_ref[1]) * inv
  h = (jnp.dot(mean, wl_ref[...], preferred_element_type=jnp.float32)
       + hr_ref[...])
  h = jnp.maximum(h, 0.0)
  out_ref[...] = (jnp.dot(h, wfc_ref[...], preferred_element_type=jnp.float32)
                  + bfc_ref[...])


_AGG_SPEC = pl.BlockSpec((NC, BM, F), lambda m: (0, m, 0))
_DEG_SPEC = pl.BlockSpec((BM, NCLS), lambda m: (m, 0))
_ROW_SPEC = pl.BlockSpec((BM, F), lambda m: (m, 0))
_W_SPEC = pl.BlockSpec((F, F), lambda m: (0, 0))
_B_SPEC = pl.BlockSpec((1, F), lambda m: (0, 0))

_tcr = pl.pallas_call(
    _tcr_body,
    grid=(GRID,),
    in_specs=[_ROW_SPEC, _W_SPEC, _B_SPEC],
    out_specs=_ROW_SPEC,
    out_shape=jax.ShapeDtypeStruct((N, F), jnp.float32),
)

_tc1 = pl.pallas_call(
    _tc1_body,
    grid=(GRID,),
    in_specs=[_AGG_SPEC, _DEG_SPEC, _ROW_SPEC, _W_SPEC],
    out_specs=_ROW_SPEC,
    out_shape=jax.ShapeDtypeStruct((N, F), jnp.float32),
)

_tc2 = pl.pallas_call(
    _tc2_body,
    grid=(GRID,),
    in_specs=[_AGG_SPEC, _DEG_SPEC, _ROW_SPEC, _W_SPEC, _W_SPEC, _B_SPEC],
    out_specs=_ROW_SPEC,
    out_shape=jax.ShapeDtypeStruct((N, F), jnp.float32),
)


def kernel(x, edge_index, W1l, W1r, b1, W2l, W2r, b2, Wfc, bfc):
  srcr = edge_index[0].astype(jnp.int32).reshape(NW, NSLAB, SL, 1, C)
  dstr = edge_index[1].astype(jnp.int32).reshape(NW, NSLAB, SL, 1, C)
  idx = jnp.concatenate([srcr, dstr], axis=3).reshape(NW * NSLAB, SL, 2, C)
  zx = jnp.zeros((NPAD, F), jnp.float32)
  zd = jnp.zeros((NPAD,), jnp.float32)

  xr = _tcr(x, W1r.T, b1.reshape(1, F))
  aggp, degp = _segsum_deg(x, idx, zx, zd)
  degt = degp.reshape(NC, NPAD).T
  h = _tc1(aggp, degt, xr, W1l.T)

  hr = _tcr(h, W2r.T, b2.reshape(1, F))
  aggp2 = _segsum(h, idx, zx, zd)
  if isinstance(aggp2, (list, tuple)):
    aggp2 = aggp2[0]
  wfc_pad = jnp.zeros((F, F), jnp.float32).at[:, :NCLS].set(Wfc.T)
  bfc_pad = jnp.zeros((1, F), jnp.float32).at[0, :NCLS].set(bfc)
  out_pad = _tc2(aggp2, degt, hr, W2l.T, wfc_pad, bfc_pad)
  return out_pad[:, :NCLS]
